# baseline (speedup 1.0000x reference)
import functools

import jax
import jax.numpy as jnp
from jax import lax
from jax.experimental import pallas as pl
from jax.experimental.pallas import tpu as pltpu
from jax.experimental.pallas import tpu_sc as plsc

NUM_NODES = 100000
NUM_EDGE_TYPES = 16
N_CAND = 64
EMBED_DIM = 64
HIDDEN = 64
EPISODE_LEN = 10
B = 1024
STEPS = EPISODE_LEN - 1
D2 = 2 * EMBED_DIM

_NC = 2
_NS = 16
_NW = _NC * _NS
_BPW = B // _NW
_ROWS_PW = _BPW * N_CAND
_CH = 128
_GRP = 4
_GROWS = _CH * _GRP
_NIT = _ROWS_PW // _GROWS


def _sc_mesh():
    return plsc.VectorSubcoreMesh(
        core_axis_name="c", subcore_axis_name="s", num_cores=_NC, num_subcores=_NS
    )


def _wid():
    return lax.axis_index("s") * _NC + lax.axis_index("c")


def _make_sc_gather(m):
    rpw = m // _NW

    @functools.partial(
        pl.kernel,
        out_type=jax.ShapeDtypeStruct((m, D2), jnp.float32),
        mesh=_sc_mesh(),
        scratch_types=[
            pltpu.VMEM((rpw,), jnp.int32),
            pltpu.VMEM((rpw, D2), jnp.float32),
            pltpu.SemaphoreType.DMA,
        ],
    )
    def gather_k(table_hbm, idx_hbm, out_hbm, idx_v, rows_v, sem):
        base = _wid() * rpw
        pltpu.sync_copy(idx_hbm.at[pl.ds(base, rpw)], idx_v)
        pltpu.async_copy(table_hbm.at[idx_v], rows_v, sem).wait()
        pltpu.sync_copy(rows_v, out_hbm.at[pl.ds(base, rpw)])

    return gather_k


@functools.cache
def _get_sc_step(goff):
    return functools.partial(
        pl.kernel,
        out_type=[
            jax.ShapeDtypeStruct((B,), jnp.int32),
            jax.ShapeDtypeStruct((B, D2), jnp.float32),
            jax.ShapeDtypeStruct((B, D2), jnp.float32),
        ],
        mesh=_sc_mesh(),
        scratch_types=[
            pltpu.VMEM((_BPW,), jnp.int32),
            pltpu.VMEM((_BPW, D2), jnp.int32),
            pltpu.VMEM((_BPW, D2), jnp.float32),
            pltpu.VMEM((_BPW, D2), jnp.float32),
            pltpu.VMEM((_ROWS_PW,), jnp.int32),
            pltpu.VMEM((2, _CH, D2), jnp.float32),
            pltpu.VMEM((_ROWS_PW,), jnp.float32),
            pltpu.VMEM((_BPW,), jnp.int32),
            pltpu.VMEM((_BPW,), jnp.int32),
            pltpu.VMEM((_BPW, D2), jnp.float32),
            pltpu.VMEM((_BPW, D2), jnp.float32),
            pltpu.SemaphoreType.DMA,
            pltpu.SemaphoreType.DMA,
        ],
        compiler_params=pltpu.CompilerParams(needs_layout_passes=False),
    )(functools.partial(_sc_step_body, goff))


def _sc_step_body(goff, cur_hbm, envb_hbm, emb_hbm, edge_hbm, xes_hbm,
                  gum_hbm, ci_out, sel_out, eo_out,
                  idx_v, bothv, xes_v, gum_v, flat_n, buf_v, pro_v,
                  ci_v, st_v, selrows, eorows, sem_a, sem_b):
    base = _wid() * _BPW
    pltpu.sync_copy(cur_hbm.at[pl.ds(base, _BPW)], idx_v)
    pltpu.sync_copy(xes_hbm.at[pl.ds(base, _BPW)], xes_v)
    pltpu.sync_copy(gum_hbm.at[pl.ds(base, _BPW)], gum_v)
    pltpu.async_copy(envb_hbm.at[idx_v], bothv, sem_a).wait()
    for i in range(_BPW):
        for k in range(N_CAND // 16):
            flat_n[pl.ds(i * N_CAND + k * 16, 16)] = bothv[i, pl.ds(k * 16, 16)]

    lanei = lax.broadcasted_iota(jnp.int32, (16,), 0)

    def _issue(chunk, slot):
        return pltpu.async_copy(
            emb_hbm.at[flat_n.at[pl.ds(chunk * _CH, _CH)]], buf_v.at[slot], sem_b)

    def _drain(slot):
        pltpu.make_async_copy(emb_hbm.at[pl.ds(0, _CH)], buf_v.at[slot], sem_b).wait()

    def _compute(slot, chunk):
        for w2 in range(2):
            w = chunk * 2 + w2
            wv = jnp.full((16,), w, jnp.int32)
            xq = [plsc.load_gather(xes_v, [wv, k * 16 + lanei]) for k in range(4)]
            for q in range(4):
                z = jnp.zeros((16,), jnp.float32)
                for j in range(16):
                    r = w2 * 64 + q * 16 + j
                    a = buf_v[slot, r, pl.ds(0, 16)] * xq[0]
                    a = a + buf_v[slot, r, pl.ds(16, 16)] * xq[1]
                    a = a + buf_v[slot, r, pl.ds(32, 16)] * xq[2]
                    a = a + buf_v[slot, r, pl.ds(48, 16)] * xq[3]
                    z = jnp.where(lanei == j, jnp.sum(a), z)
                pro_v[pl.ds(w * N_CAND + q * 16, 16)] = z

    _issue(0, 0)

    def loop_body(c2, carry):
        chunk0 = c2 * 2
        _issue(chunk0 + 1, 1)
        _drain(0)
        _compute(0, chunk0)

        @pl.when(c2 < _ROWS_PW // _CH // 2 - 1)
        def _():
            _issue(chunk0 + 2, 0)

        _drain(1)
        _compute(1, chunk0 + 1)
        return carry

    lax.fori_loop(0, _ROWS_PW // _CH // 2, loop_body, 0)

    for g in range(_BPW // 16):
        wl = g * 16 + lanei
        m = jnp.full((16,), -jnp.inf, jnp.float32)
        am = jnp.zeros((16,), jnp.int32)
        for c in range(N_CAND):
            cv = jnp.full((16,), c, jnp.int32)
            pv = plsc.load_gather(pro_v, [wl * N_CAND + c])
            gv = plsc.load_gather(gum_v, [wl, goff + cv])
            et = plsc.load_gather(bothv, [wl, N_CAND + cv])
            ev = plsc.load_gather(xes_v, [wl, N_CAND + et])
            y = (pv + ev) + gv
            upd = y > m
            m = jnp.where(upd, y, m)
            am = jnp.where(upd, cv, am)
        nv = plsc.load_gather(bothv, [wl, am])
        tv = plsc.load_gather(bothv, [wl, N_CAND + am])
        ci_v[pl.ds(g * 16, 16)] = nv
        st_v[pl.ds(g * 16, 16)] = tv

    da = pltpu.async_copy(emb_hbm.at[ci_v], selrows, sem_a)
    db = pltpu.async_copy(edge_hbm.at[st_v], eorows, sem_b)
    da.wait()
    db.wait()
    pltpu.sync_copy(ci_v, ci_out.at[pl.ds(base, _BPW)])
    pltpu.sync_copy(selrows, sel_out.at[pl.ds(base, _BPW)])
    pltpu.sync_copy(eorows, eo_out.at[pl.ds(base, _BPW)])


def _sc_step_call(cur, envb, emb128, edge128, xes, gum128, goff=0):
    return _get_sc_step(goff)(cur, envb, emb128, edge128, xes, gum128)


def _tc_core(h, q, wrefs):
    (wirT, wizT, winT, whrT, whzT, whnT,
     bir, biz, bin_, bhr, bhz, bhn,
     fc1T, fc1b, fc2T, fc2b, etabT) = wrefs
    f32 = jnp.float32
    dot = lambda a, b: jnp.dot(a, b, preferred_element_type=f32)
    i_r = dot(q, wirT[...]) + bir[...]
    i_z = dot(q, wizT[...]) + biz[...]
    i_n = dot(q, winT[...]) + bin_[...]
    h_r = dot(h, whrT[...]) + bhr[...]
    h_z = dot(h, whzT[...]) + bhz[...]
    h_n = dot(h, whnT[...]) + bhn[...]
    r = jax.nn.sigmoid(i_r + h_r)
    z = jax.nn.sigmoid(i_z + h_z)
    n = jnp.tanh(i_n + r * h_n)
    hn = (1.0 - z) * n + z * h
    hid = jnp.maximum(dot(hn, fc1T[...]) + fc1b[...], 0.0)
    x = dot(hid, fc2T[...]) + fc2b[...]
    es = dot(x, etabT[...])
    xes = jnp.concatenate([x, es, jnp.zeros((B, D2 - EMBED_DIM - NUM_EDGE_TYPES),
                                            f32)], axis=1)
    return hn, xes


def _tc_first_body(se_ref, ee_ref, *rest):
    wrefs = rest[:17]
    ho_ref, xeso_ref = rest[17:]
    se = se_ref[...]
    ee = ee_ref[...]
    hn, xes = _tc_core(se + ee, se, wrefs)
    ho_ref[...] = hn
    xeso_ref[...] = xes


def _tc_step_body(h_ref, xesp_ref, selp_ref, eop_ref, se_ref, ee_ref, *rest):
    wrefs = rest[:17]
    ho_ref, xeso_ref, relo_ref = rest[17:]
    xprev = xesp_ref[...][:, :EMBED_DIM]
    selp = selp_ref[...][:, :EMBED_DIM]
    eop = eop_ref[...][:, :EMBED_DIM]
    relo_ref[...] = jnp.sum((selp + eop) * xprev, axis=1, keepdims=True)
    q = eop + selp + se_ref[...] + ee_ref[...]
    hn, xes = _tc_core(h_ref[...], q, wrefs)
    ho_ref[...] = hn
    xeso_ref[...] = xes


def _tc_epi_body(xesp_ref, selp_ref, eop_ref, relo_ref):
    xprev = xesp_ref[...][:, :EMBED_DIM]
    selp = selp_ref[...][:, :EMBED_DIM]
    eop = eop_ref[...][:, :EMBED_DIM]
    relo_ref[...] = jnp.sum((selp + eop) * xprev, axis=1, keepdims=True)


def _f32s(shp):
    return jax.ShapeDtypeStruct(shp, jnp.float32)


def _make_tc(interpret=False):
    first = pl.pallas_call(
        _tc_first_body,
        out_shape=[_f32s((B, EMBED_DIM)), _f32s((B, D2))],
        interpret=interpret,
    )
    step = pl.pallas_call(
        _tc_step_body,
        out_shape=[_f32s((B, EMBED_DIM)), _f32s((B, D2)), _f32s((B, 1))],
        interpret=interpret,
    )
    epi = pl.pallas_call(
        _tc_epi_body,
        out_shape=[_f32s((B, 1))],
        interpret=interpret,
    )
    return first, step, epi


_TC_FIRST, _TC_STEP, _TC_EPI = _make_tc()


def kernel(start_inds, end_inds, user2item_idx, emb_table, edge_table,
           env_node_table, env_edge_table, W_ih, W_hh, b_ih, b_hh,
           fc1_w, fc1_b, fc2_w, fc2_b):
    f32 = jnp.float32
    start_inds = start_inds.astype(jnp.int32)
    end_inds = end_inds.astype(jnp.int32)

    wT = W_ih.T
    wirT, wizT, winT = wT[:, :HIDDEN], wT[:, HIDDEN:2 * HIDDEN], wT[:, 2 * HIDDEN:]
    hT = W_hh.T
    whrT, whzT, whnT = hT[:, :HIDDEN], hT[:, HIDDEN:2 * HIDDEN], hT[:, 2 * HIDDEN:]
    bir, biz, bin_ = (b_ih[:HIDDEN].reshape(1, -1),
                      b_ih[HIDDEN:2 * HIDDEN].reshape(1, -1),
                      b_ih[2 * HIDDEN:].reshape(1, -1))
    bhr, bhz, bhn = (b_hh[:HIDDEN].reshape(1, -1),
                     b_hh[HIDDEN:2 * HIDDEN].reshape(1, -1),
                     b_hh[2 * HIDDEN:].reshape(1, -1))
    fc1T, fc2T = fc1_w.T, fc2_w.T
    fc1b, fc2b = fc1_b.reshape(1, -1), fc2_b.reshape(1, -1)
    etabT = edge_table.T
    weights = (wirT, wizT, winT, whrT, whzT, whnT, bir, biz, bin_, bhr, bhz, bhn,
               fc1T, fc1b, fc2T, fc2b, etabT)

    graw = [
        jax.random.gumbel(jax.random.fold_in(jax.random.key(1), s),
                          (B, N_CAND), f32)
        for s in range(STEPS)
    ]
    graw.append(jnp.zeros((B, N_CAND), f32))
    gum = [jnp.concatenate([graw[2 * k], graw[2 * k + 1]], axis=1)
           for k in range((STEPS + 1) // 2)]

    emb128 = jnp.concatenate([emb_table, emb_table], axis=1)
    envb = jnp.concatenate([env_node_table, env_edge_table], axis=1)
    edge128 = jnp.concatenate([edge_table, jnp.zeros_like(edge_table)], axis=1)

    both = _make_sc_gather(2 * B)(
        emb128, jnp.concatenate([start_inds, end_inds]))[:, :EMBED_DIM]
    se, ee = both[:B], both[B:]

    rel_embeds = jnp.broadcast_to(
        edge_table[jnp.asarray(user2item_idx, jnp.int32)], (B, EMBED_DIM))

    cur = start_inds
    out_embeds = [se]
    out_inds = [start_inds]
    hist = [se, rel_embeds, ee]
    relav = jnp.zeros((B, 1), f32)

    h, xes = _TC_FIRST(se, ee, *weights)
    for s in range(STEPS):
        if s > 0:
            h, xes, rel_k = _TC_STEP(h, xes, sel, eo, se, ee, *weights)
            relav = relav + rel_k
        cur, sel, eo = _sc_step_call(cur, envb, emb128, edge128, xes,
                                     gum[s // 2], goff=N_CAND * (s % 2))
        out_inds.append(cur)
        out_embeds.append(sel[:, :EMBED_DIM])
        if s >= 1:
            hist.append(eo[:, :EMBED_DIM])
            hist.append(sel[:, :EMBED_DIM])

    rel_k, = _TC_EPI(xes, sel, eo)
    relav = relav + rel_k

    return (jnp.stack(out_embeds, 0), jnp.stack(out_inds, 0), se, ee,
            jnp.stack(hist, 0), relav)


# speedup vs baseline: 1.0548x; 1.0548x over previous
import functools

import jax
import jax.numpy as jnp
from jax import lax
from jax.experimental import pallas as pl
from jax.experimental.pallas import tpu as pltpu
from jax.experimental.pallas import tpu_sc as plsc

NUM_NODES = 100000
NUM_EDGE_TYPES = 16
N_CAND = 64
EMBED_DIM = 64
HIDDEN = 64
EPISODE_LEN = 10
B = 1024
STEPS = EPISODE_LEN - 1
D2 = 2 * EMBED_DIM

_NC = 2
_NS = 16
_NW = _NC * _NS
_BPW = B // _NW
_ROWS_PW = _BPW * N_CAND
_CH = 128
_GRP = 4
_GROWS = _CH * _GRP
_NIT = _ROWS_PW // _GROWS


def _sc_mesh():
    return plsc.VectorSubcoreMesh(
        core_axis_name="c", subcore_axis_name="s", num_cores=_NC, num_subcores=_NS
    )


def _wid():
    return lax.axis_index("s") * _NC + lax.axis_index("c")


def _make_sc_gather(m):
    rpw = m // _NW

    @functools.partial(
        pl.kernel,
        out_type=jax.ShapeDtypeStruct((m, D2), jnp.float32),
        mesh=_sc_mesh(),
        scratch_types=[
            pltpu.VMEM((rpw,), jnp.int32),
            pltpu.VMEM((rpw, D2), jnp.float32),
            pltpu.SemaphoreType.DMA,
        ],
    )
    def gather_k(table_hbm, idx_hbm, out_hbm, idx_v, rows_v, sem):
        base = _wid() * rpw
        pltpu.sync_copy(idx_hbm.at[pl.ds(base, rpw)], idx_v)
        pltpu.async_copy(table_hbm.at[idx_v], rows_v, sem).wait()
        pltpu.sync_copy(rows_v, out_hbm.at[pl.ds(base, rpw)])

    return gather_k


@functools.cache
def _get_sc_step(goff):
    return functools.partial(
        pl.kernel,
        out_type=[
            jax.ShapeDtypeStruct((B,), jnp.int32),
            jax.ShapeDtypeStruct((B, D2), jnp.float32),
            jax.ShapeDtypeStruct((B, D2), jnp.float32),
        ],
        mesh=_sc_mesh(),
        scratch_types=[
            pltpu.VMEM((_BPW,), jnp.int32),
            pltpu.VMEM((_BPW, D2), jnp.int32),
            pltpu.VMEM((_BPW, D2), jnp.float32),
            pltpu.VMEM((_BPW, D2), jnp.float32),
            pltpu.VMEM((_ROWS_PW,), jnp.int32),
            pltpu.VMEM((2, _CH, D2), jnp.float32),
            pltpu.VMEM((_ROWS_PW,), jnp.float32),
            pltpu.VMEM((_BPW,), jnp.int32),
            pltpu.VMEM((_BPW,), jnp.int32),
            pltpu.VMEM((_BPW, D2), jnp.float32),
            pltpu.VMEM((_BPW, D2), jnp.float32),
            pltpu.SemaphoreType.DMA,
            pltpu.SemaphoreType.DMA,
        ],
        compiler_params=pltpu.CompilerParams(needs_layout_passes=False),
    )(functools.partial(_sc_step_body, goff))


def _sc_step_body(goff, cur_hbm, envb_hbm, emb_hbm, edge_hbm, xes_hbm,
                  gum_hbm, ci_out, sel_out, eo_out,
                  idx_v, bothv, xes_v, gum_v, flat_n, buf_v, pro_v,
                  ci_v, st_v, selrows, eorows, sem_a, sem_b):
    base = _wid() * _BPW
    pltpu.sync_copy(cur_hbm.at[pl.ds(base, _BPW)], idx_v)
    pltpu.sync_copy(xes_hbm.at[pl.ds(base, _BPW)], xes_v)
    pltpu.sync_copy(gum_hbm.at[pl.ds(base, _BPW)], gum_v)
    pltpu.async_copy(envb_hbm.at[idx_v], bothv, sem_a).wait()
    for i in range(_BPW):
        for k in range(N_CAND // 16):
            flat_n[pl.ds(i * N_CAND + k * 16, 16)] = bothv[i, pl.ds(k * 16, 16)]

    lanei = lax.broadcasted_iota(jnp.int32, (16,), 0)

    def _issue(chunk, slot):
        return pltpu.async_copy(
            emb_hbm.at[flat_n.at[pl.ds(chunk * _CH, _CH)]], buf_v.at[slot], sem_b)

    def _drain(slot):
        pltpu.make_async_copy(emb_hbm.at[pl.ds(0, _CH)], buf_v.at[slot], sem_b).wait()

    def _compute(slot, chunk):
        for w2 in range(2):
            w = chunk * 2 + w2
            wv = jnp.full((16,), w, jnp.int32)
            xq = [plsc.load_gather(xes_v, [wv, k * 16 + lanei]) for k in range(4)]
            for q in range(4):
                z = jnp.zeros((16,), jnp.float32)
                for j in range(16):
                    r = w2 * 64 + q * 16 + j
                    a = buf_v[slot, r, pl.ds(0, 16)] * xq[0]
                    a = a + buf_v[slot, r, pl.ds(16, 16)] * xq[1]
                    a = a + buf_v[slot, r, pl.ds(32, 16)] * xq[2]
                    a = a + buf_v[slot, r, pl.ds(48, 16)] * xq[3]
                    z = jnp.where(lanei == j, jnp.sum(a), z)
                pro_v[pl.ds(w * N_CAND + q * 16, 16)] = z

    _issue(0, 0)

    def loop_body(c2, carry):
        chunk0 = c2 * 2
        _issue(chunk0 + 1, 1)
        _drain(0)
        _compute(0, chunk0)

        @pl.when(c2 < _ROWS_PW // _CH // 2 - 1)
        def _():
            _issue(chunk0 + 2, 0)

        _drain(1)
        _compute(1, chunk0 + 1)
        return carry

    lax.fori_loop(0, _ROWS_PW // _CH // 2, loop_body, 0)

    for g in range(_BPW // 16):
        wl = g * 16 + lanei
        m = jnp.full((16,), -jnp.inf, jnp.float32)
        am = jnp.zeros((16,), jnp.int32)
        for c in range(N_CAND):
            cv = jnp.full((16,), c, jnp.int32)
            pv = plsc.load_gather(pro_v, [wl * N_CAND + c])
            gv = plsc.load_gather(gum_v, [wl, goff + cv])
            et = plsc.load_gather(bothv, [wl, N_CAND + cv])
            ev = plsc.load_gather(xes_v, [wl, N_CAND + et])
            y = (pv + ev) + gv
            upd = y > m
            m = jnp.where(upd, y, m)
            am = jnp.where(upd, cv, am)
        nv = plsc.load_gather(bothv, [wl, am])
        tv = plsc.load_gather(bothv, [wl, N_CAND + am])
        ci_v[pl.ds(g * 16, 16)] = nv
        st_v[pl.ds(g * 16, 16)] = tv

    da = pltpu.async_copy(emb_hbm.at[ci_v], selrows, sem_a)
    db = pltpu.async_copy(edge_hbm.at[st_v], eorows, sem_b)
    da.wait()
    db.wait()
    pltpu.sync_copy(ci_v, ci_out.at[pl.ds(base, _BPW)])
    pltpu.sync_copy(selrows, sel_out.at[pl.ds(base, _BPW)])
    pltpu.sync_copy(eorows, eo_out.at[pl.ds(base, _BPW)])


def _sc_step_call(cur, envb, emb128, edge128, xes, gum128, goff=0):
    return _get_sc_step(goff)(cur, envb, emb128, edge128, xes, gum128)


def _tc_core(h, q, wrefs):
    (wirT, wizT, winT, whrT, whzT, whnT,
     bir, biz, bin_, bhr, bhz, bhn,
     fc1T, fc1b, fc2T, fc2b, etabT) = wrefs
    f32 = jnp.float32
    dot = lambda a, b: jnp.dot(a, b, preferred_element_type=f32)
    i_r = dot(q, wirT[...]) + bir[...]
    i_z = dot(q, wizT[...]) + biz[...]
    i_n = dot(q, winT[...]) + bin_[...]
    h_r = dot(h, whrT[...]) + bhr[...]
    h_z = dot(h, whzT[...]) + bhz[...]
    h_n = dot(h, whnT[...]) + bhn[...]
    r = jax.nn.sigmoid(i_r + h_r)
    z = jax.nn.sigmoid(i_z + h_z)
    n = jnp.tanh(i_n + r * h_n)
    hn = (1.0 - z) * n + z * h
    hid = jnp.maximum(dot(hn, fc1T[...]) + fc1b[...], 0.0)
    x = dot(hid, fc2T[...]) + fc2b[...]
    es = dot(x, etabT[...])
    xes = jnp.concatenate([x, es, jnp.zeros((B, D2 - EMBED_DIM - NUM_EDGE_TYPES),
                                            f32)], axis=1)
    return hn, xes


def _tc_first_body(se_ref, ee_ref, *rest):
    wrefs = rest[:17]
    ho_ref, xeso_ref = rest[17:]
    se = se_ref[...]
    ee = ee_ref[...]
    hn, xes = _tc_core(se + ee, se, wrefs)
    ho_ref[...] = hn
    xeso_ref[...] = xes


def _tc_step_body(h_ref, xesp_ref, selp_ref, eop_ref, se_ref, ee_ref, *rest):
    wrefs = rest[:17]
    ho_ref, xeso_ref, relo_ref = rest[17:]
    xprev = xesp_ref[...][:, :EMBED_DIM]
    selp = selp_ref[...][:, :EMBED_DIM]
    eop = eop_ref[...][:, :EMBED_DIM]
    relo_ref[...] = jnp.sum((selp + eop) * xprev, axis=1, keepdims=True)
    q = eop + selp + se_ref[...] + ee_ref[...]
    hn, xes = _tc_core(h_ref[...], q, wrefs)
    ho_ref[...] = hn
    xeso_ref[...] = xes


def _tc_epi_body(xesp_ref, selp_ref, eop_ref, relo_ref):
    xprev = xesp_ref[...][:, :EMBED_DIM]
    selp = selp_ref[...][:, :EMBED_DIM]
    eop = eop_ref[...][:, :EMBED_DIM]
    relo_ref[...] = jnp.sum((selp + eop) * xprev, axis=1, keepdims=True)


def _f32s(shp):
    return jax.ShapeDtypeStruct(shp, jnp.float32)


def _make_tc(interpret=False):
    first = pl.pallas_call(
        _tc_first_body,
        out_shape=[_f32s((B, EMBED_DIM)), _f32s((B, D2))],
        interpret=interpret,
    )
    step = pl.pallas_call(
        _tc_step_body,
        out_shape=[_f32s((B, EMBED_DIM)), _f32s((B, D2)), _f32s((B, 1))],
        interpret=interpret,
    )
    epi = pl.pallas_call(
        _tc_epi_body,
        out_shape=[_f32s((B, 1))],
        interpret=interpret,
    )
    return first, step, epi


_TC_FIRST, _TC_STEP, _TC_EPI = _make_tc()


def kernel(start_inds, end_inds, user2item_idx, emb_table, edge_table,
           env_node_table, env_edge_table, W_ih, W_hh, b_ih, b_hh,
           fc1_w, fc1_b, fc2_w, fc2_b):
    f32 = jnp.float32
    start_inds = start_inds.astype(jnp.int32)
    end_inds = end_inds.astype(jnp.int32)

    wT = W_ih.T
    wirT, wizT, winT = wT[:, :HIDDEN], wT[:, HIDDEN:2 * HIDDEN], wT[:, 2 * HIDDEN:]
    hT = W_hh.T
    whrT, whzT, whnT = hT[:, :HIDDEN], hT[:, HIDDEN:2 * HIDDEN], hT[:, 2 * HIDDEN:]
    bir, biz, bin_ = (b_ih[:HIDDEN].reshape(1, -1),
                      b_ih[HIDDEN:2 * HIDDEN].reshape(1, -1),
                      b_ih[2 * HIDDEN:].reshape(1, -1))
    bhr, bhz, bhn = (b_hh[:HIDDEN].reshape(1, -1),
                     b_hh[HIDDEN:2 * HIDDEN].reshape(1, -1),
                     b_hh[2 * HIDDEN:].reshape(1, -1))
    fc1T, fc2T = fc1_w.T, fc2_w.T
    fc1b, fc2b = fc1_b.reshape(1, -1), fc2_b.reshape(1, -1)
    etabT = edge_table.T
    weights = (wirT, wizT, winT, whrT, whzT, whnT, bir, biz, bin_, bhr, bhz, bhn,
               fc1T, fc1b, fc2T, fc2b, etabT)

    graw = [
        jax.random.gumbel(jax.random.fold_in(jax.random.key(1), s),
                          (B, N_CAND), f32)
        for s in range(STEPS)
    ]
    graw.append(jnp.zeros((B, N_CAND), f32))
    gum = [jnp.concatenate([graw[2 * k], graw[2 * k + 1]], axis=1)
           for k in range((STEPS + 1) // 2)]

    emb128 = jnp.concatenate([emb_table, jnp.zeros_like(emb_table)], axis=1)
    envb = jnp.concatenate([env_node_table, env_edge_table], axis=1)
    edge128 = jnp.concatenate([edge_table, jnp.zeros_like(edge_table)], axis=1)

    both = _make_sc_gather(2 * B)(
        emb128, jnp.concatenate([start_inds, end_inds]))[:, :EMBED_DIM]
    se, ee = both[:B], both[B:]

    rel_embeds = jnp.broadcast_to(
        edge_table[jnp.asarray(user2item_idx, jnp.int32)], (B, EMBED_DIM))

    cur = start_inds
    out_embeds = [se]
    out_inds = [start_inds]
    hist = [se, rel_embeds, ee]
    relav = jnp.zeros((B, 1), f32)

    h, xes = _TC_FIRST(se, ee, *weights)
    for s in range(STEPS):
        if s > 0:
            h, xes, rel_k = _TC_STEP(h, xes, sel, eo, se, ee, *weights)
            relav = relav + rel_k
        cur, sel, eo = _sc_step_call(cur, envb, emb128, edge128, xes,
                                     gum[s // 2], goff=N_CAND * (s % 2))
        out_inds.append(cur)
        out_embeds.append(sel[:, :EMBED_DIM])
        if s >= 1:
            hist.append(eo[:, :EMBED_DIM])
            hist.append(sel[:, :EMBED_DIM])

    rel_k, = _TC_EPI(xes, sel, eo)
    relav = relav + rel_k

    return (jnp.stack(out_embeds, 0), jnp.stack(out_inds, 0), se, ee,
            jnp.stack(hist, 0), relav)


# speedup vs baseline: 1.0564x; 1.0015x over previous
import functools

import jax
import jax.numpy as jnp
from jax import lax
from jax.experimental import pallas as pl
from jax.experimental.pallas import tpu as pltpu
from jax.experimental.pallas import tpu_sc as plsc

NUM_NODES = 100000
NUM_EDGE_TYPES = 16
N_CAND = 64
EMBED_DIM = 64
HIDDEN = 64
EPISODE_LEN = 10
B = 1024
STEPS = EPISODE_LEN - 1
D2 = 2 * EMBED_DIM

_NC = 2
_NS = 16
_NW = _NC * _NS
_BPW = B // _NW
_ROWS_PW = _BPW * N_CAND
_CH = 128
_GRP = 4
_GROWS = _CH * _GRP
_NIT = _ROWS_PW // _GROWS


def _sc_mesh():
    return plsc.VectorSubcoreMesh(
        core_axis_name="c", subcore_axis_name="s", num_cores=_NC, num_subcores=_NS
    )


def _wid():
    return lax.axis_index("s") * _NC + lax.axis_index("c")


def _make_sc_gather(m):
    rpw = m // _NW

    @functools.partial(
        pl.kernel,
        out_type=jax.ShapeDtypeStruct((m, D2), jnp.float32),
        mesh=_sc_mesh(),
        scratch_types=[
            pltpu.VMEM((rpw,), jnp.int32),
            pltpu.VMEM((rpw, D2), jnp.float32),
            pltpu.SemaphoreType.DMA,
        ],
    )
    def gather_k(table_hbm, idx_hbm, out_hbm, idx_v, rows_v, sem):
        base = _wid() * rpw
        pltpu.sync_copy(idx_hbm.at[pl.ds(base, rpw)], idx_v)
        pltpu.async_copy(table_hbm.at[idx_v], rows_v, sem).wait()
        pltpu.sync_copy(rows_v, out_hbm.at[pl.ds(base, rpw)])

    return gather_k


@functools.cache
def _get_sc_step(goff):
    return functools.partial(
        pl.kernel,
        out_type=[
            jax.ShapeDtypeStruct((B,), jnp.int32),
            jax.ShapeDtypeStruct((B, D2), jnp.float32),
            jax.ShapeDtypeStruct((B, D2), jnp.float32),
        ],
        mesh=_sc_mesh(),
        scratch_types=[
            pltpu.VMEM((_BPW,), jnp.int32),
            pltpu.VMEM((_BPW, D2), jnp.int32),
            pltpu.VMEM((_BPW, D2), jnp.float32),
            pltpu.VMEM((_BPW, D2), jnp.float32),
            pltpu.VMEM((_ROWS_PW,), jnp.int32),
            pltpu.VMEM((2, _CH, D2), jnp.float32),
            pltpu.VMEM((_ROWS_PW,), jnp.float32),
            pltpu.VMEM((_BPW,), jnp.int32),
            pltpu.VMEM((_BPW,), jnp.int32),
            pltpu.VMEM((_BPW, D2), jnp.float32),
            pltpu.VMEM((_BPW, D2), jnp.float32),
            pltpu.SemaphoreType.DMA,
            pltpu.SemaphoreType.DMA,
        ],
        compiler_params=pltpu.CompilerParams(needs_layout_passes=False),
    )(functools.partial(_sc_step_body, goff))


def _sc_step_body(goff, cur_hbm, envb_hbm, emb_hbm, edge_hbm, xes_hbm,
                  gum_hbm, ci_out, sel_out, eo_out,
                  idx_v, bothv, xes_v, gum_v, flat_n, buf_v, pro_v,
                  ci_v, st_v, selrows, eorows, sem_a, sem_b):
    base = _wid() * _BPW
    pltpu.sync_copy(cur_hbm.at[pl.ds(base, _BPW)], idx_v)
    pltpu.sync_copy(xes_hbm.at[pl.ds(base, _BPW)], xes_v)
    pltpu.sync_copy(gum_hbm.at[pl.ds(base, _BPW)], gum_v)
    pltpu.async_copy(envb_hbm.at[idx_v], bothv, sem_a).wait()
    for i in range(_BPW):
        for k in range(N_CAND // 16):
            flat_n[pl.ds(i * N_CAND + k * 16, 16)] = bothv[i, pl.ds(k * 16, 16)]

    lanei = lax.broadcasted_iota(jnp.int32, (16,), 0)

    def _issue(chunk, slot):
        return pltpu.async_copy(
            emb_hbm.at[flat_n.at[pl.ds(chunk * _CH, _CH)]], buf_v.at[slot], sem_b)

    def _drain(slot):
        pltpu.make_async_copy(emb_hbm.at[pl.ds(0, _CH)], buf_v.at[slot], sem_b).wait()

    def _compute(slot, chunk):
        for w2 in range(2):
            w = chunk * 2 + w2
            wv = jnp.full((16,), w, jnp.int32)
            xq = [plsc.load_gather(xes_v, [wv, k * 16 + lanei]) for k in range(4)]
            for q in range(4):
                z = jnp.zeros((16,), jnp.float32)
                for j in range(16):
                    r = w2 * 64 + q * 16 + j
                    a = buf_v[slot, r, pl.ds(0, 16)] * xq[0]
                    a = a + buf_v[slot, r, pl.ds(16, 16)] * xq[1]
                    a = a + buf_v[slot, r, pl.ds(32, 16)] * xq[2]
                    a = a + buf_v[slot, r, pl.ds(48, 16)] * xq[3]
                    z = jnp.where(lanei == j, jnp.sum(a), z)
                pro_v[pl.ds(w * N_CAND + q * 16, 16)] = z

    _issue(0, 0)

    def loop_body(c2, carry):
        chunk0 = c2 * 2
        _issue(chunk0 + 1, 1)
        _drain(0)
        _compute(0, chunk0)

        @pl.when(c2 < _ROWS_PW // _CH // 2 - 1)
        def _():
            _issue(chunk0 + 2, 0)

        _drain(1)
        _compute(1, chunk0 + 1)
        return carry

    lax.fori_loop(0, _ROWS_PW // _CH // 2, loop_body, 0)

    for g in range(_BPW // 16):
        wl = g * 16 + lanei
        m = jnp.full((16,), -jnp.inf, jnp.float32)
        am = jnp.zeros((16,), jnp.int32)
        for c in range(N_CAND):
            cv = jnp.full((16,), c, jnp.int32)
            pv = plsc.load_gather(pro_v, [wl * N_CAND + c])
            gv = plsc.load_gather(gum_v, [wl, goff + cv])
            et = plsc.load_gather(bothv, [wl, N_CAND + cv])
            ev = plsc.load_gather(xes_v, [wl, N_CAND + et])
            y = (pv + ev) + gv
            upd = y > m
            m = jnp.where(upd, y, m)
            am = jnp.where(upd, cv, am)
        nv = plsc.load_gather(bothv, [wl, am])
        tv = plsc.load_gather(bothv, [wl, N_CAND + am])
        ci_v[pl.ds(g * 16, 16)] = nv
        st_v[pl.ds(g * 16, 16)] = tv

    da = pltpu.async_copy(emb_hbm.at[ci_v], selrows, sem_a)
    db = pltpu.async_copy(edge_hbm.at[st_v], eorows, sem_b)
    da.wait()
    db.wait()
    pltpu.sync_copy(ci_v, ci_out.at[pl.ds(base, _BPW)])
    pltpu.sync_copy(selrows, sel_out.at[pl.ds(base, _BPW)])
    pltpu.sync_copy(eorows, eo_out.at[pl.ds(base, _BPW)])


def _sc_step_call(cur, envb, emb128, edge128, xes, gum128, goff=0):
    return _get_sc_step(goff)(cur, envb, emb128, edge128, xes, gum128)


def _tc_core(h, q, wrefs):
    (wirT, wizT, winT, whrT, whzT, whnT,
     bir, biz, bin_, bhr, bhz, bhn,
     fc1T, fc1b, fc2T, fc2b, etabT) = wrefs
    f32 = jnp.float32
    dot = lambda a, b: jnp.dot(a, b, preferred_element_type=f32)
    i_r = dot(q, wirT[...]) + bir[...]
    i_z = dot(q, wizT[...]) + biz[...]
    i_n = dot(q, winT[...]) + bin_[...]
    h_r = dot(h, whrT[...]) + bhr[...]
    h_z = dot(h, whzT[...]) + bhz[...]
    h_n = dot(h, whnT[...]) + bhn[...]
    r = jax.nn.sigmoid(i_r + h_r)
    z = jax.nn.sigmoid(i_z + h_z)
    n = jnp.tanh(i_n + r * h_n)
    hn = (1.0 - z) * n + z * h
    hid = jnp.maximum(dot(hn, fc1T[...]) + fc1b[...], 0.0)
    x = dot(hid, fc2T[...]) + fc2b[...]
    es = dot(x, etabT[...])
    xes = jnp.concatenate([x, es, jnp.zeros((B, D2 - EMBED_DIM - NUM_EDGE_TYPES),
                                            f32)], axis=1)
    return hn, xes


def _tc_first_body(se_ref, ee_ref, *rest):
    wrefs = rest[:17]
    ho_ref, xeso_ref = rest[17:]
    se = se_ref[...]
    ee = ee_ref[...]
    hn, xes = _tc_core(se + ee, se, wrefs)
    ho_ref[...] = hn
    xeso_ref[...] = xes


def _tc_step_body(h_ref, xesp_ref, selp_ref, eop_ref, se_ref, ee_ref, *rest):
    wrefs = rest[:17]
    ho_ref, xeso_ref, relo_ref = rest[17:]
    xprev = xesp_ref[...][:, :EMBED_DIM]
    selp = selp_ref[...][:, :EMBED_DIM]
    eop = eop_ref[...][:, :EMBED_DIM]
    relo_ref[...] = jnp.sum((selp + eop) * xprev, axis=1, keepdims=True)
    q = eop + selp + se_ref[...] + ee_ref[...]
    hn, xes = _tc_core(h_ref[...], q, wrefs)
    ho_ref[...] = hn
    xeso_ref[...] = xes


def _tc_epi_body(xesp_ref, selp_ref, eop_ref, relo_ref):
    xprev = xesp_ref[...][:, :EMBED_DIM]
    selp = selp_ref[...][:, :EMBED_DIM]
    eop = eop_ref[...][:, :EMBED_DIM]
    relo_ref[...] = jnp.sum((selp + eop) * xprev, axis=1, keepdims=True)


def _f32s(shp):
    return jax.ShapeDtypeStruct(shp, jnp.float32)


def _make_tc(interpret=False):
    first = pl.pallas_call(
        _tc_first_body,
        out_shape=[_f32s((B, EMBED_DIM)), _f32s((B, D2))],
        interpret=interpret,
    )
    step = pl.pallas_call(
        _tc_step_body,
        out_shape=[_f32s((B, EMBED_DIM)), _f32s((B, D2)), _f32s((B, 1))],
        interpret=interpret,
    )
    epi = pl.pallas_call(
        _tc_epi_body,
        out_shape=[_f32s((B, 1))],
        interpret=interpret,
    )
    return first, step, epi


_TC_FIRST, _TC_STEP, _TC_EPI = _make_tc()


def kernel(start_inds, end_inds, user2item_idx, emb_table, edge_table,
           env_node_table, env_edge_table, W_ih, W_hh, b_ih, b_hh,
           fc1_w, fc1_b, fc2_w, fc2_b):
    f32 = jnp.float32
    start_inds = start_inds.astype(jnp.int32)
    end_inds = end_inds.astype(jnp.int32)

    wT = W_ih.T
    wirT, wizT, winT = wT[:, :HIDDEN], wT[:, HIDDEN:2 * HIDDEN], wT[:, 2 * HIDDEN:]
    hT = W_hh.T
    whrT, whzT, whnT = hT[:, :HIDDEN], hT[:, HIDDEN:2 * HIDDEN], hT[:, 2 * HIDDEN:]
    bir, biz, bin_ = (b_ih[:HIDDEN].reshape(1, -1),
                      b_ih[HIDDEN:2 * HIDDEN].reshape(1, -1),
                      b_ih[2 * HIDDEN:].reshape(1, -1))
    bhr, bhz, bhn = (b_hh[:HIDDEN].reshape(1, -1),
                     b_hh[HIDDEN:2 * HIDDEN].reshape(1, -1),
                     b_hh[2 * HIDDEN:].reshape(1, -1))
    fc1T, fc2T = fc1_w.T, fc2_w.T
    fc1b, fc2b = fc1_b.reshape(1, -1), fc2_b.reshape(1, -1)
    etabT = edge_table.T
    weights = (wirT, wizT, winT, whrT, whzT, whnT, bir, biz, bin_, bhr, bhz, bhn,
               fc1T, fc1b, fc2T, fc2b, etabT)

    graw = [
        jax.random.gumbel(jax.random.fold_in(jax.random.key(1), s),
                          (B, N_CAND), f32)
        for s in range(STEPS)
    ]
    graw.append(jnp.zeros((B, N_CAND), f32))
    gum = [jnp.concatenate([graw[2 * k], graw[2 * k + 1]], axis=1)
           for k in range((STEPS + 1) // 2)]

    emb128 = jnp.concatenate([emb_table, jnp.zeros_like(emb_table)], axis=1)
    envb = jnp.concatenate([env_node_table, env_edge_table], axis=1)
    edge128 = jnp.concatenate([edge_table, jnp.zeros_like(edge_table)], axis=1)

    both = _make_sc_gather(2 * B)(
        emb128, jnp.concatenate([start_inds, end_inds]))
    se128, ee128 = both[:B], both[B:]
    se, ee = se128[:, :EMBED_DIM], ee128[:, :EMBED_DIM]

    rel_embeds = jnp.broadcast_to(
        edge128[jnp.asarray(user2item_idx, jnp.int32)], (B, D2))

    cur = start_inds
    out_embeds = [se128]
    out_inds = [start_inds]
    hist = [se128, rel_embeds, ee128]
    relav = jnp.zeros((B, 1), f32)

    h, xes = _TC_FIRST(se, ee, *weights)
    for s in range(STEPS):
        if s > 0:
            h, xes, rel_k = _TC_STEP(h, xes, sel, eo, se, ee, *weights)
            relav = relav + rel_k
        cur, sel, eo = _sc_step_call(cur, envb, emb128, edge128, xes,
                                     gum[s // 2], goff=N_CAND * (s % 2))
        out_inds.append(cur)
        out_embeds.append(sel)
        if s >= 1:
            hist.append(eo)
            hist.append(sel)

    rel_k, = _TC_EPI(xes, sel, eo)
    relav = relav + rel_k

    return (jnp.stack(out_embeds, 0)[:, :, :EMBED_DIM],
            jnp.stack(out_inds, 0), se, ee,
            jnp.stack(hist, 0)[:, :, :EMBED_DIM], relav)
